# baseline (speedup 1.0000x reference)
import jax
import jax.numpy as jnp
from jax import lax
from jax.experimental import pallas as pl
from jax.experimental.pallas import tpu as pltpu
from jax.experimental.pallas import tpu_sc as plsc

N = 10000
E = 320000
D = 128

NC, NS, LANES = 2, 16, 16
NW = NC * NS
DP = 16
NA = 10240
CH = 128
NCHT = 160
EP = NW * NCHT * CH // 2
RING = 8
ROWT = N // NS

AGG_C0, AGG_C1 = 160, 0
DEG_C0, DEG_C1 = 96, 64

_MESH = plsc.VectorSubcoreMesh(
    core_axis_name="c", subcore_axis_name="s", num_cores=NC, num_subcores=NS)


def _sc_agg_body(src_hbm, dst_hbm, tbl_hbm, zero_hbm, out_hbm,
                 idx_s, idx_d, bufs, acc, gsems, ssems):
    c = lax.axis_index("c")
    s = lax.axis_index("s")
    pltpu.sync_copy(zero_hbm.at[pl.ds(s * ROWT, ROWT)],
                    acc.at[pl.ds(s * ROWT, ROWT)])
    plsc.subcore_barrier()

    def run(cpt, base):
        pltpu.sync_copy(src_hbm.at[pl.ds(base, cpt)], idx_s.at[pl.ds(0, cpt)])
        pltpu.sync_copy(dst_hbm.at[pl.ds(base, cpt)], idx_d.at[pl.ds(0, cpt)])
        for b in range(RING):
            pltpu.async_copy(tbl_hbm.at[idx_s.at[b]], bufs[b], gsems[b])

        def group(t, carry):
            j0 = RING * t
            for b in range(RING):
                pltpu.make_async_copy(
                    zero_hbm.at[pl.ds(0, CH)], bufs[b], gsems[b]).wait()
                pltpu.async_copy(bufs[b], acc.at[idx_d.at[j0 + b]], ssems[b],
                                 add=True)
            for b in range(RING):
                pltpu.make_async_copy(
                    zero_hbm.at[pl.ds(0, CH)], bufs[b], ssems[b]).wait()

                @pl.when(j0 + b + RING < cpt)
                def _():
                    pltpu.async_copy(tbl_hbm.at[idx_s.at[j0 + b + RING]],
                                     bufs[b], gsems[b])

            return carry

        lax.fori_loop(0, cpt // RING, group, 0)

    @pl.when(c == 0)
    def _():
        run(AGG_C0, s * AGG_C0)

    if AGG_C1:
        @pl.when(c == 1)
        def _():
            run(AGG_C1, NS * AGG_C0 + s * AGG_C1)

    plsc.subcore_barrier()
    pltpu.sync_copy(acc.at[pl.ds(s * ROWT, ROWT)],
                    out_hbm.at[pl.ds((c * N) + s * ROWT, ROWT)])


def _sc_deg_body(dst_hbm, ones_hbm, zero_hbm, out_hbm, idx_d, rows, acc, sem):
    c = lax.axis_index("c")
    s = lax.axis_index("s")
    pltpu.sync_copy(zero_hbm.at[pl.ds(s * ROWT, ROWT)],
                    acc.at[pl.ds(s * ROWT, ROWT)])
    pltpu.sync_copy(ones_hbm.at[pl.ds(0, CH)], rows)
    plsc.subcore_barrier()

    def run(cpt, base):
        pltpu.sync_copy(dst_hbm.at[pl.ds(base, cpt)], idx_d.at[pl.ds(0, cpt)])

        def fire(j, carry):
            pltpu.async_copy(rows, acc.at[idx_d.at[j]], sem, add=True)
            return carry

        lax.fori_loop(0, cpt, fire, 0)

        def drain(j, carry):
            pltpu.make_async_copy(zero_hbm.at[pl.ds(0, CH)], rows, sem).wait()
            return carry

        lax.fori_loop(0, cpt, drain, 0)

    @pl.when(c == 0)
    def _():
        run(DEG_C0, s * DEG_C0)

    @pl.when(c == 1)
    def _():
        run(DEG_C1, NS * DEG_C0 + s * DEG_C1)

    plsc.subcore_barrier()
    pltpu.sync_copy(acc.at[pl.ds(s * ROWT, ROWT)],
                    out_hbm.at[pl.ds((c * N) + s * ROWT, ROWT)])


_SC_PARAMS = pltpu.CompilerParams(use_tc_tiling_on_sc=False)

_sc_agg = pl.kernel(
    _sc_agg_body,
    out_type=jax.ShapeDtypeStruct((NC * N, DP), jnp.float32),
    mesh=_MESH,
    compiler_params=_SC_PARAMS,
    scratch_types=[
        pltpu.VMEM((AGG_C0, CH), jnp.int32),
        pltpu.VMEM((AGG_C0, CH), jnp.int32),
        [pltpu.VMEM((CH, DP), jnp.float32) for _ in range(RING)],
        pltpu.VMEM_SHARED((NA, DP), jnp.float32),
        [pltpu.SemaphoreType.DMA for _ in range(RING)],
        [pltpu.SemaphoreType.DMA for _ in range(RING)],
    ],
)

_sc_deg = pl.kernel(
    _sc_deg_body,
    out_type=jax.ShapeDtypeStruct((NC * N, DP), jnp.float32),
    mesh=_MESH,
    compiler_params=_SC_PARAMS,
    scratch_types=[
        pltpu.VMEM((DEG_C0, CH), jnp.int32),
        pltpu.VMEM((CH, DP), jnp.float32),
        pltpu.VMEM_SHARED((NA, DP), jnp.float32),
        pltpu.SemaphoreType.DMA,
    ],
)


BN = 1000


def _dot(a, b):
    return jnp.dot(a, b, preferred_element_type=jnp.float32,
                   precision=lax.Precision.HIGHEST)


def _dis(degp_ref):
    return lax.rsqrt(degp_ref[0] + degp_ref[1] + 1.0)


def _p1_body(x_ref, w1_ref, degp_ref, o_ref):
    h = jnp.dot(x_ref[...], w1_ref[...], preferred_element_type=jnp.float32)
    o_ref[...] = h * _dis(degp_ref)


def _z1_body(acc_ref, p1_ref, degp_ref, b1_ref, o_ref):
    dis = _dis(degp_ref)
    agg = acc_ref[0] + acc_ref[1] + p1_ref[...]
    z = jnp.maximum(dis * agg + b1_ref[...], 0.0)
    zb = z.astype(jnp.bfloat16).astype(jnp.float32)
    o_ref[...] = dis * zb


def _head_body(acc_ref, p2_ref, degp_ref, w2_ref, b2_ref, w3_ref, b3_ref,
               w4_ref, b4_ref, o_ref):
    dis = _dis(degp_ref)
    g = dis * (acc_ref[0] + acc_ref[1] + p2_ref[...])
    h2 = _dot(g, w2_ref[...]) + b2_ref[...]
    h3 = jnp.maximum(
        jnp.dot(h2, w3_ref[...], preferred_element_type=jnp.float32)
        + b3_ref[...], 0.0)
    o_ref[...] = (jnp.dot(h3, w4_ref[...], preferred_element_type=jnp.float32)
                  + b4_ref[...])


def _row_spec(d):
    return pl.BlockSpec((BN, d), lambda i: (i, 0))


def _pair_spec():
    return pl.BlockSpec((2, BN, DP), lambda i: (0, i, 0))


def _full_spec(r, c):
    return pl.BlockSpec((r, c), lambda i: (0, 0))


_GRID = (N // BN,)

_p1_call = pl.pallas_call(
    _p1_body,
    grid=_GRID,
    in_specs=[_row_spec(D), _full_spec(D, DP), _pair_spec()],
    out_specs=_row_spec(DP),
    out_shape=jax.ShapeDtypeStruct((N, DP), jnp.float32),
)

_z1_call = pl.pallas_call(
    _z1_body,
    grid=_GRID,
    in_specs=[_pair_spec(), _row_spec(DP), _pair_spec(), _full_spec(1, DP)],
    out_specs=_row_spec(DP),
    out_shape=jax.ShapeDtypeStruct((N, DP), jnp.float32),
)

_head_call = pl.pallas_call(
    _head_body,
    grid=_GRID,
    in_specs=[_pair_spec(), _row_spec(DP), _pair_spec(), _full_spec(DP, 256),
              _full_spec(1, 256), _full_spec(256, 256), _full_spec(1, 256),
              _full_spec(256, 1), _full_spec(1, 1)],
    out_specs=_row_spec(1),
    out_shape=jax.ShapeDtypeStruct((N, 1), jnp.float32),
)


def kernel(x, edge_index, W1, b1, W2, b2, W3, b3, W4, b4):
    f32 = jnp.float32
    pad_s = jnp.zeros((EP - E,), dtype=jnp.int32)
    pad_d = jnp.full((EP - E,), N, dtype=jnp.int32)
    src = jnp.concatenate([edge_index[0].astype(jnp.int32), pad_s]).reshape(
        NS * NCHT, CH)
    dst = jnp.concatenate([edge_index[1].astype(jnp.int32), pad_d]).reshape(
        NS * NCHT, CH)
    w1p = jnp.zeros((D, DP), f32).at[:, :8].set(W1)
    b1p = jnp.zeros((1, DP), f32).at[0, :8].set(b1)
    w2p = jnp.zeros((DP, 256), f32).at[:8].set(
        W2.astype(jnp.bfloat16).astype(f32))
    zeros_tbl = jnp.zeros((N, DP), f32)
    ones_tbl = jnp.ones((CH, DP), f32)

    degp = _sc_deg(dst, ones_tbl, zeros_tbl).reshape(2, N, DP)

    p1 = _p1_call(x, w1p, degp)

    acc1 = _sc_agg(src, dst, p1, zeros_tbl).reshape(2, N, DP)

    p2 = _z1_call(acc1, p1, degp, b1p)

    acc2 = _sc_agg(src, dst, p2, zeros_tbl).reshape(2, N, DP)

    out = _head_call(acc2, p2, degp, w2p, b2.reshape(1, 256),
                     W3, b3.reshape(1, 256), W4, b4.reshape(1, 1))
    return out


# speedup vs baseline: 1.1353x; 1.1353x over previous
import jax
import jax.numpy as jnp
from jax import lax
from jax.experimental import pallas as pl
from jax.experimental.pallas import tpu as pltpu
from jax.experimental.pallas import tpu_sc as plsc

N = 10000
E = 320000
D = 128

NC, NS, LANES = 2, 16, 16
NW = NC * NS
DP = 16
NA = 10240
CH = 128
NCHT = 160
EP = NW * NCHT * CH // 2
RING = 8
ROWT = N // NS

AGG_C0, AGG_C1 = 112, 48
DEG_C0, DEG_C1 = 96, 64

_MESH = plsc.VectorSubcoreMesh(
    core_axis_name="c", subcore_axis_name="s", num_cores=NC, num_subcores=NS)


def _sc_agg_body(src_hbm, dst_hbm, tbl_hbm, zero_hbm, out_hbm,
                 idx_s, idx_d, bufs, acc, gsems, ssems):
    c = lax.axis_index("c")
    s = lax.axis_index("s")
    pltpu.sync_copy(zero_hbm.at[pl.ds(s * ROWT, ROWT)],
                    acc.at[pl.ds(s * ROWT, ROWT)])
    plsc.subcore_barrier()

    def run(cpt, base):
        pltpu.sync_copy(src_hbm.at[pl.ds(base, cpt)], idx_s.at[pl.ds(0, cpt)])
        pltpu.sync_copy(dst_hbm.at[pl.ds(base, cpt)], idx_d.at[pl.ds(0, cpt)])
        for b in range(RING):
            pltpu.async_copy(tbl_hbm.at[idx_s.at[b]], bufs[b], gsems[b])

        def group(t, carry):
            j0 = RING * t
            for b in range(RING):
                pltpu.make_async_copy(
                    zero_hbm.at[pl.ds(0, CH)], bufs[b], gsems[b]).wait()
                pltpu.async_copy(bufs[b], acc.at[idx_d.at[j0 + b]], ssems[b],
                                 add=True)
            for b in range(RING):
                pltpu.make_async_copy(
                    zero_hbm.at[pl.ds(0, CH)], bufs[b], ssems[b]).wait()

                @pl.when(j0 + b + RING < cpt)
                def _():
                    pltpu.async_copy(tbl_hbm.at[idx_s.at[j0 + b + RING]],
                                     bufs[b], gsems[b])

            return carry

        lax.fori_loop(0, cpt // RING, group, 0)

    @pl.when(c == 0)
    def _():
        run(AGG_C0, s * AGG_C0)

    if AGG_C1:
        @pl.when(c == 1)
        def _():
            run(AGG_C1, NS * AGG_C0 + s * AGG_C1)

    plsc.subcore_barrier()
    pltpu.sync_copy(acc.at[pl.ds(s * ROWT, ROWT)],
                    out_hbm.at[pl.ds((c * N) + s * ROWT, ROWT)])


def _sc_deg_body(dst_hbm, ones_hbm, zero_hbm, out_hbm, idx_d, rows, acc, sem):
    c = lax.axis_index("c")
    s = lax.axis_index("s")
    pltpu.sync_copy(zero_hbm.at[pl.ds(s * ROWT, ROWT)],
                    acc.at[pl.ds(s * ROWT, ROWT)])
    pltpu.sync_copy(ones_hbm.at[pl.ds(0, CH)], rows)
    plsc.subcore_barrier()

    def run(cpt, base):
        pltpu.sync_copy(dst_hbm.at[pl.ds(base, cpt)], idx_d.at[pl.ds(0, cpt)])

        def fire(j, carry):
            pltpu.async_copy(rows, acc.at[idx_d.at[j]], sem, add=True)
            return carry

        lax.fori_loop(0, cpt, fire, 0)

        def drain(j, carry):
            pltpu.make_async_copy(zero_hbm.at[pl.ds(0, CH)], rows, sem).wait()
            return carry

        lax.fori_loop(0, cpt, drain, 0)

    @pl.when(c == 0)
    def _():
        run(DEG_C0, s * DEG_C0)

    @pl.when(c == 1)
    def _():
        run(DEG_C1, NS * DEG_C0 + s * DEG_C1)

    plsc.subcore_barrier()
    pltpu.sync_copy(acc.at[pl.ds(s * ROWT, ROWT)],
                    out_hbm.at[pl.ds((c * N) + s * ROWT, ROWT)])


_SC_PARAMS = pltpu.CompilerParams(use_tc_tiling_on_sc=False)

_sc_agg = pl.kernel(
    _sc_agg_body,
    out_type=jax.ShapeDtypeStruct((NC * N, DP), jnp.float32),
    mesh=_MESH,
    compiler_params=_SC_PARAMS,
    scratch_types=[
        pltpu.VMEM((AGG_C0, CH), jnp.int32),
        pltpu.VMEM((AGG_C0, CH), jnp.int32),
        [pltpu.VMEM((CH, DP), jnp.float32) for _ in range(RING)],
        pltpu.VMEM_SHARED((NA, DP), jnp.float32),
        [pltpu.SemaphoreType.DMA for _ in range(RING)],
        [pltpu.SemaphoreType.DMA for _ in range(RING)],
    ],
)

_sc_deg = pl.kernel(
    _sc_deg_body,
    out_type=jax.ShapeDtypeStruct((NC * N, DP), jnp.float32),
    mesh=_MESH,
    compiler_params=_SC_PARAMS,
    scratch_types=[
        pltpu.VMEM((DEG_C0, CH), jnp.int32),
        pltpu.VMEM((CH, DP), jnp.float32),
        pltpu.VMEM_SHARED((NA, DP), jnp.float32),
        pltpu.SemaphoreType.DMA,
    ],
)


BN = 1000


def _dot(a, b):
    return jnp.dot(a, b, preferred_element_type=jnp.float32,
                   precision=lax.Precision.HIGHEST)


def _dis(degp_ref):
    return lax.rsqrt(degp_ref[0] + degp_ref[1] + 1.0)


def _p1_body(x_ref, w1_ref, degp_ref, o_ref):
    h = jnp.dot(x_ref[...], w1_ref[...], preferred_element_type=jnp.float32)
    o_ref[...] = h * _dis(degp_ref)


def _z1_body(acc_ref, p1_ref, degp_ref, b1_ref, o_ref):
    dis = _dis(degp_ref)
    agg = acc_ref[0] + acc_ref[1] + p1_ref[...]
    z = jnp.maximum(dis * agg + b1_ref[...], 0.0)
    zb = z.astype(jnp.bfloat16).astype(jnp.float32)
    o_ref[...] = dis * zb


def _head_body(acc_ref, p2_ref, degp_ref, w2_ref, b2_ref, w3_ref, b3_ref,
               w4_ref, b4_ref, o_ref):
    dis = _dis(degp_ref)
    g = dis * (acc_ref[0] + acc_ref[1] + p2_ref[...])
    h2 = _dot(g, w2_ref[...]) + b2_ref[...]
    h3 = jnp.maximum(
        jnp.dot(h2, w3_ref[...], preferred_element_type=jnp.float32)
        + b3_ref[...], 0.0)
    o_ref[...] = (jnp.dot(h3, w4_ref[...], preferred_element_type=jnp.float32)
                  + b4_ref[...])


def _row_spec(d):
    return pl.BlockSpec((BN, d), lambda i: (i, 0))


def _pair_spec():
    return pl.BlockSpec((2, BN, DP), lambda i: (0, i, 0))


def _full_spec(r, c):
    return pl.BlockSpec((r, c), lambda i: (0, 0))


_GRID = (N // BN,)

_p1_call = pl.pallas_call(
    _p1_body,
    grid=_GRID,
    in_specs=[_row_spec(D), _full_spec(D, DP), _pair_spec()],
    out_specs=_row_spec(DP),
    out_shape=jax.ShapeDtypeStruct((N, DP), jnp.float32),
)

_z1_call = pl.pallas_call(
    _z1_body,
    grid=_GRID,
    in_specs=[_pair_spec(), _row_spec(DP), _pair_spec(), _full_spec(1, DP)],
    out_specs=_row_spec(DP),
    out_shape=jax.ShapeDtypeStruct((N, DP), jnp.float32),
)

_head_call = pl.pallas_call(
    _head_body,
    grid=_GRID,
    in_specs=[_pair_spec(), _row_spec(DP), _pair_spec(), _full_spec(DP, 256),
              _full_spec(1, 256), _full_spec(256, 256), _full_spec(1, 256),
              _full_spec(256, 1), _full_spec(1, 1)],
    out_specs=_row_spec(1),
    out_shape=jax.ShapeDtypeStruct((N, 1), jnp.float32),
)


def kernel(x, edge_index, W1, b1, W2, b2, W3, b3, W4, b4):
    f32 = jnp.float32
    pad_s = jnp.zeros((EP - E,), dtype=jnp.int32)
    pad_d = jnp.full((EP - E,), N, dtype=jnp.int32)
    src = jnp.concatenate([edge_index[0].astype(jnp.int32), pad_s]).reshape(
        NS * NCHT, CH)
    dst = jnp.concatenate([edge_index[1].astype(jnp.int32), pad_d]).reshape(
        NS * NCHT, CH)
    w1p = jnp.zeros((D, DP), f32).at[:, :8].set(W1)
    b1p = jnp.zeros((1, DP), f32).at[0, :8].set(b1)
    w2p = jnp.zeros((DP, 256), f32).at[:8].set(
        W2.astype(jnp.bfloat16).astype(f32))
    zeros_tbl = jnp.zeros((N, DP), f32)
    ones_tbl = jnp.ones((CH, DP), f32)

    degp = _sc_deg(dst, ones_tbl, zeros_tbl).reshape(2, N, DP)

    p1 = _p1_call(x, w1p, degp)

    acc1 = _sc_agg(src, dst, p1, zeros_tbl).reshape(2, N, DP)

    p2 = _z1_call(acc1, p1, degp, b1p)

    acc2 = _sc_agg(src, dst, p2, zeros_tbl).reshape(2, N, DP)

    out = _head_call(acc2, p2, degp, w2p, b2.reshape(1, 256),
                     W3, b3.reshape(1, 256), W4, b4.reshape(1, 1))
    return out


# speedup vs baseline: 1.1660x; 1.0270x over previous
import jax
import jax.numpy as jnp
from jax import lax
from jax.experimental import pallas as pl
from jax.experimental.pallas import tpu as pltpu
from jax.experimental.pallas import tpu_sc as plsc

N = 10000
E = 320000
D = 128

NC, NS, LANES = 2, 16, 16
NW = NC * NS
DP = 16
NA = 10240
CH = 128
NCHT = 160
EP = NW * NCHT * CH // 2
RING = 8
ROWT = N // NS

AGG_C0, AGG_C1 = 112, 48
DEG_C0, DEG_C1 = 96, 64

_MESH = plsc.VectorSubcoreMesh(
    core_axis_name="c", subcore_axis_name="s", num_cores=NC, num_subcores=NS)


def _sc_agg_body(src_hbm, dst_hbm, tbl_hbm, zero_hbm, out_hbm,
                 idx_s, idx_d, bufs, acc, gsems, ssems):
    c = lax.axis_index("c")
    s = lax.axis_index("s")
    pltpu.sync_copy(zero_hbm.at[pl.ds(s * ROWT, ROWT)],
                    acc.at[pl.ds(s * ROWT, ROWT)])
    plsc.subcore_barrier()

    def run(cpt, base):
        pltpu.sync_copy(src_hbm.at[pl.ds(base, cpt)], idx_s.at[pl.ds(0, cpt)])
        pltpu.sync_copy(dst_hbm.at[pl.ds(base, cpt)], idx_d.at[pl.ds(0, cpt)])
        for b in range(RING):
            pltpu.async_copy(tbl_hbm.at[idx_s.at[b]], bufs[b], gsems[b])

        def group(t, carry):
            j0 = RING * t
            for b in range(RING):
                pltpu.make_async_copy(
                    zero_hbm.at[pl.ds(0, CH)], bufs[b], gsems[b]).wait()
                pltpu.async_copy(bufs[b], acc.at[idx_d.at[j0 + b]], ssems[b],
                                 add=True)
            for b in range(RING):
                pltpu.make_async_copy(
                    zero_hbm.at[pl.ds(0, CH)], bufs[b], ssems[b]).wait()

                @pl.when(j0 + b + RING < cpt)
                def _():
                    pltpu.async_copy(tbl_hbm.at[idx_s.at[j0 + b + RING]],
                                     bufs[b], gsems[b])

            return carry

        lax.fori_loop(0, cpt // RING, group, 0)

    @pl.when(c == 0)
    def _():
        run(AGG_C0, s * AGG_C0)

    if AGG_C1:
        @pl.when(c == 1)
        def _():
            run(AGG_C1, NS * AGG_C0 + s * AGG_C1)

    plsc.subcore_barrier()
    pltpu.sync_copy(acc.at[pl.ds(s * ROWT, ROWT)],
                    out_hbm.at[pl.ds((c * N) + s * ROWT, ROWT)])


def _sc_deg_body(dst_hbm, ones_hbm, zero_hbm, out_hbm, idx_d, rows, acc, sem):
    c = lax.axis_index("c")
    s = lax.axis_index("s")
    pltpu.sync_copy(zero_hbm.at[pl.ds(s * ROWT, ROWT)],
                    acc.at[pl.ds(s * ROWT, ROWT)])
    pltpu.sync_copy(ones_hbm.at[pl.ds(0, CH)], rows)
    plsc.subcore_barrier()

    def run(cpt, base):
        pltpu.sync_copy(dst_hbm.at[pl.ds(base, cpt)], idx_d.at[pl.ds(0, cpt)])

        def fire(j, carry):
            pltpu.async_copy(rows, acc.at[idx_d.at[j]], sem, add=True)
            return carry

        lax.fori_loop(0, cpt, fire, 0)

        def drain(j, carry):
            pltpu.make_async_copy(zero_hbm.at[pl.ds(0, CH)], rows, sem).wait()
            return carry

        lax.fori_loop(0, cpt, drain, 0)

    @pl.when(c == 0)
    def _():
        run(DEG_C0, s * DEG_C0)

    @pl.when(c == 1)
    def _():
        run(DEG_C1, NS * DEG_C0 + s * DEG_C1)

    plsc.subcore_barrier()
    pltpu.sync_copy(acc.at[pl.ds(s * ROWT, ROWT)],
                    out_hbm.at[pl.ds((c * N) + s * ROWT, ROWT)])


_SC_PARAMS = pltpu.CompilerParams(use_tc_tiling_on_sc=False)

_sc_agg = pl.kernel(
    _sc_agg_body,
    out_type=jax.ShapeDtypeStruct((NC * N, DP), jnp.float32),
    mesh=_MESH,
    compiler_params=_SC_PARAMS,
    scratch_types=[
        pltpu.VMEM((AGG_C0, CH), jnp.int32),
        pltpu.VMEM((AGG_C0, CH), jnp.int32),
        [pltpu.VMEM((CH, DP), jnp.float32) for _ in range(RING)],
        pltpu.VMEM_SHARED((NA, DP), jnp.float32),
        [pltpu.SemaphoreType.DMA for _ in range(RING)],
        [pltpu.SemaphoreType.DMA for _ in range(RING)],
    ],
)

_sc_deg = pl.kernel(
    _sc_deg_body,
    out_type=jax.ShapeDtypeStruct((NC * N, DP), jnp.float32),
    mesh=_MESH,
    compiler_params=_SC_PARAMS,
    scratch_types=[
        pltpu.VMEM((DEG_C0, CH), jnp.int32),
        pltpu.VMEM((CH, DP), jnp.float32),
        pltpu.VMEM_SHARED((NA, DP), jnp.float32),
        pltpu.SemaphoreType.DMA,
    ],
)


BN = 2000


def _dot(a, b):
    return jnp.dot(a, b, preferred_element_type=jnp.float32,
                   precision=lax.Precision.HIGHEST)


def _dis(degp_ref):
    return lax.rsqrt(degp_ref[0] + degp_ref[1] + 1.0)


def _p1_body(x_ref, w1_ref, degp_ref, o_ref):
    h = jnp.dot(x_ref[...], w1_ref[...], preferred_element_type=jnp.float32)
    o_ref[...] = h * _dis(degp_ref)


def _z1_body(acc_ref, p1_ref, degp_ref, b1_ref, o_ref):
    dis = _dis(degp_ref)
    agg = acc_ref[0] + acc_ref[1] + p1_ref[...]
    z = jnp.maximum(dis * agg + b1_ref[...], 0.0)
    zb = z.astype(jnp.bfloat16).astype(jnp.float32)
    o_ref[...] = dis * zb


def _head_body(acc_ref, p2_ref, degp_ref, w2_ref, b2_ref, w3_ref, b3_ref,
               w4_ref, b4_ref, o_ref):
    dis = _dis(degp_ref)
    g = dis * (acc_ref[0] + acc_ref[1] + p2_ref[...])
    h2 = _dot(g, w2_ref[...]) + b2_ref[...]
    h3 = jnp.maximum(
        jnp.dot(h2, w3_ref[...], preferred_element_type=jnp.float32)
        + b3_ref[...], 0.0)
    o_ref[...] = (jnp.dot(h3, w4_ref[...], preferred_element_type=jnp.float32)
                  + b4_ref[...])


def _row_spec(d):
    return pl.BlockSpec((BN, d), lambda i: (i, 0))


def _pair_spec():
    return pl.BlockSpec((2, BN, DP), lambda i: (0, i, 0))


def _full_spec(r, c):
    return pl.BlockSpec((r, c), lambda i: (0, 0))


_GRID = (N // BN,)

_p1_call = pl.pallas_call(
    _p1_body,
    grid=_GRID,
    in_specs=[_row_spec(D), _full_spec(D, DP), _pair_spec()],
    out_specs=_row_spec(DP),
    out_shape=jax.ShapeDtypeStruct((N, DP), jnp.float32),
)

_z1_call = pl.pallas_call(
    _z1_body,
    grid=_GRID,
    in_specs=[_pair_spec(), _row_spec(DP), _pair_spec(), _full_spec(1, DP)],
    out_specs=_row_spec(DP),
    out_shape=jax.ShapeDtypeStruct((N, DP), jnp.float32),
)

_head_call = pl.pallas_call(
    _head_body,
    grid=_GRID,
    in_specs=[_pair_spec(), _row_spec(DP), _pair_spec(), _full_spec(DP, 256),
              _full_spec(1, 256), _full_spec(256, 256), _full_spec(1, 256),
              _full_spec(256, 1), _full_spec(1, 1)],
    out_specs=_row_spec(1),
    out_shape=jax.ShapeDtypeStruct((N, 1), jnp.float32),
)


def kernel(x, edge_index, W1, b1, W2, b2, W3, b3, W4, b4):
    f32 = jnp.float32
    pad_s = jnp.zeros((EP - E,), dtype=jnp.int32)
    pad_d = jnp.full((EP - E,), N, dtype=jnp.int32)
    src = jnp.concatenate([edge_index[0].astype(jnp.int32), pad_s]).reshape(
        NS * NCHT, CH)
    dst = jnp.concatenate([edge_index[1].astype(jnp.int32), pad_d]).reshape(
        NS * NCHT, CH)
    w1p = jnp.zeros((D, DP), f32).at[:, :8].set(W1)
    b1p = jnp.zeros((1, DP), f32).at[0, :8].set(b1)
    w2p = jnp.zeros((DP, 256), f32).at[:8].set(
        W2.astype(jnp.bfloat16).astype(f32))
    zeros_tbl = jnp.zeros((N, DP), f32)
    ones_tbl = jnp.ones((CH, DP), f32)

    degp = _sc_deg(dst, ones_tbl, zeros_tbl).reshape(2, N, DP)

    p1 = _p1_call(x, w1p, degp)

    acc1 = _sc_agg(src, dst, p1, zeros_tbl).reshape(2, N, DP)

    p2 = _z1_call(acc1, p1, degp, b1p)

    acc2 = _sc_agg(src, dst, p2, zeros_tbl).reshape(2, N, DP)

    out = _head_call(acc2, p2, degp, w2p, b2.reshape(1, 256),
                     W3, b3.reshape(1, 256), W4, b4.reshape(1, 1))
    return out
